# baseline (speedup 1.0000x reference)
import jax
import jax.numpy as jnp
from jax.experimental import pallas as pl
from jax.experimental.pallas import tpu as pltpu

_K = 64


def _stage_roll(x, iota, j, asc):
    bit = (iota & j) != 0
    fwd = pltpu.roll(x, j, 0)
    bwd = pltpu.roll(x, x.shape[0] - j, 0)
    partner = jnp.where(bit, fwd, bwd)
    mx = jnp.maximum(x, partner)
    mn = jnp.minimum(x, partner)
    keep_max = bit == asc
    return jnp.where(keep_max, mx, mn)


def _stage_static(x, j, k):
    n, c = x.shape
    if k >= 2 * n:
        v = x.reshape(-1, 2, j, c)
        a, b = v[:, 0], v[:, 1]
        return jnp.concatenate(
            [jnp.maximum(a, b)[:, None], jnp.minimum(a, b)[:, None]],
            axis=1).reshape(n, c)
    v = x.reshape(-1, 2, k // (2 * j), 2, j, c)
    a0, a1 = v[:, 0, :, 0], v[:, 0, :, 1]
    b0, b1 = v[:, 1, :, 0], v[:, 1, :, 1]
    na = jnp.concatenate(
        [jnp.minimum(a0, a1)[:, :, None], jnp.maximum(a0, a1)[:, :, None]],
        axis=2)
    nb = jnp.concatenate(
        [jnp.maximum(b0, b1)[:, :, None], jnp.minimum(b0, b1)[:, :, None]],
        axis=2)
    return jnp.concatenate([na[:, None], nb[:, None]], axis=1).reshape(n, c)


def _topk_body(x_ref, o_ref):
    x = x_ref[0]
    n, c = x.shape
    iota = jax.lax.broadcasted_iota(jnp.int32, (n, 1), 0)
    for k in (2, 4, 8, 16, 32, 64):
        asc = (iota & k) == 0
        j = k // 2
        while j:
            if j >= 8:
                x = _stage_static(x, j, k)
            else:
                x = _stage_roll(x, iota, j, asc)
            j //= 2
    while n > _K:
        x = x.reshape(n // 128, 2, _K, c)
        x = jnp.maximum(x[:, 0], x[:, 1]).reshape(n // 2, c)
        n //= 2
        it = jax.lax.broadcasted_iota(jnp.int32, (n, 1), 0)
        k = 64 if n > _K else 4 * n
        asc = ((it & k) == 0) if n > _K else jnp.zeros((n, 1), jnp.bool_)
        for j in (32, 16, 8, 4, 2, 1):
            if j >= 8:
                x = _stage_static(x, j, k)
            else:
                x = _stage_roll(x, it, j, asc)
    o_ref[0] = x


def kernel(x):
    b, s, d = x.shape
    c = 512
    return pl.pallas_call(
        _topk_body,
        grid=(b, d // c),
        in_specs=[pl.BlockSpec((1, s, c), lambda i, j: (i, 0, j))],
        out_specs=pl.BlockSpec((1, _K, c), lambda i, j: (i, 0, j)),
        out_shape=jax.ShapeDtypeStruct((b, _K, d), x.dtype),
        compiler_params=pltpu.CompilerParams(
            dimension_semantics=("parallel", "parallel")),
    )(x)


# speedup vs baseline: 1.0534x; 1.0534x over previous
import jax
import jax.numpy as jnp
from jax.experimental import pallas as pl
from jax.experimental.pallas import tpu as pltpu

_K = 64


def _stage_roll(x, bit, keep_max, j):
    fwd = jnp.roll(x, j, axis=0)
    bwd = jnp.roll(x, -j, axis=0)
    partner = jnp.where(bit, fwd, bwd)
    mx = jnp.maximum(x, partner)
    mn = jnp.minimum(x, partner)
    return jnp.where(keep_max, mx, mn)


def _stage_static(x, j, k):
    n, c = x.shape
    if k >= 2 * n:
        v = x.reshape(-1, 2, j, c)
        a, b = v[:, 0], v[:, 1]
        return jnp.concatenate(
            [jnp.maximum(a, b)[:, None], jnp.minimum(a, b)[:, None]],
            axis=1).reshape(n, c)
    v = x.reshape(-1, 2, k // (2 * j), 2, j, c)
    a0, a1 = v[:, 0, :, 0], v[:, 0, :, 1]
    b0, b1 = v[:, 1, :, 0], v[:, 1, :, 1]
    na = jnp.concatenate(
        [jnp.minimum(a0, a1)[:, :, None], jnp.maximum(a0, a1)[:, :, None]],
        axis=2)
    nb = jnp.concatenate(
        [jnp.maximum(b0, b1)[:, :, None], jnp.minimum(b0, b1)[:, :, None]],
        axis=2)
    return jnp.concatenate([na[:, None], nb[:, None]], axis=1).reshape(n, c)


def _topk_body(x_ref, o_ref):
    x = x_ref[0]
    n, c = x.shape
    iota = jax.lax.broadcasted_iota(jnp.int32, (n, 1), 0)
    bit = {j: (iota & j) != 0 for j in (1, 2, 4)}
    km = {}
    for k in (2, 4, 8, 16, 32, 64):
        asc = (iota & k) == 0
        for j in (1, 2, 4):
            if j < k:
                km[(j, k)] = bit[j] == asc
    for j in (1, 2, 4):
        km[(j, 0)] = jnp.logical_not(bit[j])
    for k in (2, 4, 8, 16, 32, 64):
        j = k // 2
        while j:
            if j >= 8:
                x = _stage_static(x, j, k)
            else:
                x = _stage_roll(x, bit[j][:n], km[(j, k)][:n], j)
            j //= 2
    while n > _K:
        x = x.reshape(n // 128, 2, _K, c)
        x = jnp.maximum(x[:, 0], x[:, 1]).reshape(n // 2, c)
        n //= 2
        k = 64 if n > _K else 0
        for j in (32, 16, 8, 4, 2, 1):
            if j >= 8:
                x = _stage_static(x, j, k if k else 4 * n)
            else:
                x = _stage_roll(x, bit[j][:n], km[(j, k)][:n], j)
    o_ref[0] = x


def kernel(x):
    b, s, d = x.shape
    c = 512
    return pl.pallas_call(
        _topk_body,
        grid=(b, d // c),
        in_specs=[pl.BlockSpec((1, s, c), lambda i, j: (i, 0, j))],
        out_specs=pl.BlockSpec((1, _K, c), lambda i, j: (i, 0, j)),
        out_shape=jax.ShapeDtypeStruct((b, _K, d), x.dtype),
        compiler_params=pltpu.CompilerParams(
            dimension_semantics=("parallel", "parallel")),
    )(x)
